# Initial kernel scaffold; baseline (speedup 1.0000x reference)
import jax
import jax.numpy as jnp
from jax.experimental import pallas as pl


def kernel(slot_ids, piece_ids, orientations, corner_slot_emb, corner_piece_emb, corner_orient_emb, edge_slot_emb, edge_piece_emb, edge_orient_emb, proj_W, proj_b):
    raise NotImplementedError("write your pallas kernel here")


# speedup vs baseline: 9.0071x; 9.0071x over previous
import jax
import jax.numpy as jnp
from jax.experimental import pallas as pl


_R = 512


def _tc_body(slot_ref, piece_ref, or_ref, tab_ref, w_ref, b_ref, out_ref):
    lut = jax.lax.dot_general(
        tab_ref[:], w_ref[:], (((1,), (1,)), ((), ())),
        preferred_element_type=jnp.float32)
    bias = b_ref[:]
    rows = slot_ref.shape[0]
    col = jax.lax.broadcasted_iota(jnp.int32, (rows, 48), 1)
    for t in range(20):
        so, po, oo = (0, 8, 16) if t < 8 else (19, 31, 43)
        s = slot_ref[:, t:t + 1] + so
        p = piece_ref[:, t:t + 1] + po
        o = or_ref[:, t:t + 1] + oo
        onehot = ((col == s) | (col == p) | (col == o)).astype(jnp.float32)
        res = jax.lax.dot_general(
            onehot, lut, (((1,), (0,)), ((), ())),
            preferred_element_type=jnp.float32) + bias
        out_ref[:, t, :] = res


def kernel(slot_ids, piece_ids, orientations, corner_slot_emb,
           corner_piece_emb, corner_orient_emb, edge_slot_emb, edge_piece_emb,
           edge_orient_emb, proj_W, proj_b):
    bsz = slot_ids.shape[0]
    tab = jnp.zeros((48, 128), jnp.float32)
    tab = tab.at[0:8, 0:42].set(corner_slot_emb)
    tab = tab.at[8:16, 42:84].set(corner_piece_emb)
    tab = tab.at[16:19, 84:128].set(corner_orient_emb)
    tab = tab.at[19:31, 0:42].set(edge_slot_emb)
    tab = tab.at[31:43, 42:84].set(edge_piece_emb)
    tab = tab.at[43:45, 84:128].set(edge_orient_emb)
    bias = proj_b.reshape(1, 256)

    return pl.pallas_call(
        _tc_body,
        grid=(bsz // _R,),
        in_specs=[
            pl.BlockSpec((_R, 20), lambda i: (i, 0)),
            pl.BlockSpec((_R, 20), lambda i: (i, 0)),
            pl.BlockSpec((_R, 20), lambda i: (i, 0)),
            pl.BlockSpec((48, 128), lambda i: (0, 0)),
            pl.BlockSpec((256, 128), lambda i: (0, 0)),
            pl.BlockSpec((1, 256), lambda i: (0, 0)),
        ],
        out_specs=pl.BlockSpec((_R, 20, 256), lambda i: (i, 0, 0)),
        out_shape=jax.ShapeDtypeStruct((bsz, 20, 256), jnp.float32),
    )(slot_ids, piece_ids, orientations, tab, proj_W, bias)
